# baseline (speedup 1.0000x reference)
import functools

import jax
import jax.numpy as jnp
from jax import lax
from jax.experimental import pallas as pl
from jax.experimental.pallas import tpu as pltpu
from jax.experimental.pallas import tpu_sc as plsc

EMB = 128
HID = 64

_NC = 2
_NS = 16
_NW = _NC * _NS


def _mlp_body(nsub, a0_ref, a1_ref, w10_ref, w11_ref, b1_ref, w2_ref,
              b2_ref, o_ref):
    w10 = w10_ref[...]
    w11 = w11_ref[...]
    b1 = b1_ref[...]
    w2 = w2_ref[...]
    b2 = b2_ref[...]
    for s in range(nsub):
        c0 = a0_ref[0, :, s:s + 1]
        c1 = a1_ref[0, :, s:s + 1]
        h = jnp.maximum(c0 * w10 + c1 * w11 + b1, 0.0)
        o_ref[s * 128:(s + 1) * 128, :] = (
            jax.lax.dot_general(h, w2, (((1,), (0,)), ((), ())),
                                preferred_element_type=jnp.float32)
            + b2
        )


def _mlp(a0, a1, W1, b1, W2, b2):
    grid, _, nsub = a0.shape
    rows_per_block = nsub * 128
    n = grid * rows_per_block
    return pl.pallas_call(
        functools.partial(_mlp_body, nsub),
        grid=(grid,),
        in_specs=[
            pl.BlockSpec((1, 128, nsub), lambda i: (i, 0, 0)),
            pl.BlockSpec((1, 128, nsub), lambda i: (i, 0, 0)),
            pl.BlockSpec((1, HID), lambda i: (0, 0)),
            pl.BlockSpec((1, HID), lambda i: (0, 0)),
            pl.BlockSpec((1, HID), lambda i: (0, 0)),
            pl.BlockSpec((HID, EMB), lambda i: (0, 0)),
            pl.BlockSpec((1, EMB), lambda i: (0, 0)),
        ],
        out_specs=pl.BlockSpec((rows_per_block, EMB), lambda i: (i, 0)),
        out_shape=jax.ShapeDtypeStruct((n, EMB), jnp.float32),
    )(a0, a1, W1[0:1, :], W1[1:2, :], b1.reshape(1, HID), W2,
      b2.reshape(1, EMB))


def _make_gather(total, chunk=128, nslot=4):
    per_w = total // _NW
    n_ch = per_w // chunk
    rounds = n_ch // nslot
    mesh = plsc.VectorSubcoreMesh(core_axis_name="c", subcore_axis_name="s")

    scratch = ([pltpu.VMEM((per_w,), jnp.int32)]
               + [pltpu.VMEM((chunk, EMB), jnp.float32)
                  for _ in range(nslot)]
               + [pltpu.SemaphoreType.DMA for _ in range(2 * nslot)])

    @functools.partial(
        pl.kernel,
        out_type=jax.ShapeDtypeStruct((total, EMB), jnp.float32),
        mesh=mesh,
        scratch_types=scratch,
    )
    def gather_k(idx_hbm, table_hbm, out_hbm, idx_v, *rest):
        bufs = rest[:nslot]
        gsems = rest[nslot:2 * nslot]
        ssems = rest[2 * nslot:]
        wid = lax.axis_index("s") * _NC + lax.axis_index("c")
        base = wid * per_w
        pltpu.sync_copy(idx_hbm.at[pl.ds(base, per_w)], idx_v)

        def g_sync(c, j):
            pltpu.async_copy(table_hbm.at[idx_v.at[pl.ds(c * chunk, chunk)]],
                             bufs[j], gsems[j]).wait()

        def s_start(c, j):
            pltpu.async_copy(bufs[j],
                             out_hbm.at[pl.ds(base + c * chunk, chunk)],
                             ssems[j])

        def s_wait(c, j):
            pltpu.make_async_copy(
                bufs[j], out_hbm.at[pl.ds(base + c * chunk, chunk)],
                ssems[j]).wait()

        def body(k, carry):
            for j in range(nslot):
                c = k * nslot + j

                @pl.when(k > 0)
                def _():
                    s_wait(c - nslot, j)

                g_sync(c, j)
                s_start(c, j)

            return carry

        lax.fori_loop(0, rounds, body, 0)
        for j in range(nslot):
            s_wait(n_ch - nslot + j, j)

    return gather_k


def kernel(x, solutions, visited_time, W1, b1, W2, b2, pattern):
    bs, seq, nd = x.shape
    total = bs * seq
    idx = visited_time.reshape(total).astype(jnp.int32)
    rows_per_block = 2048
    grid = total // rows_per_block
    nsub = rows_per_block // 128
    a0 = x[:, :, 0].reshape(grid, nsub, 128).transpose(0, 2, 1)
    a1 = x[:, :, 1].reshape(grid, nsub, 128).transpose(0, 2, 1)
    emb = _mlp(a0, a1, W1, b1, W2, b2).reshape(bs, seq, EMB)
    pos = _make_gather(total)(idx, pattern).reshape(bs, seq, EMB)
    return (emb, pos)


# speedup vs baseline: 1.2241x; 1.2241x over previous
import functools

import jax
import jax.numpy as jnp
from jax import lax
from jax.experimental import pallas as pl
from jax.experimental.pallas import tpu as pltpu
from jax.experimental.pallas import tpu_sc as plsc

EMB = 128
HID = 64

_NC = 2
_NS = 16
_NW = _NC * _NS


def _mlp_body(nsub, a0_ref, a1_ref, w10_ref, w11_ref, b1_ref, w2_ref,
              b2_ref, o_ref):
    w10 = w10_ref[...]
    w11 = w11_ref[...]
    b1 = b1_ref[...]
    w2 = w2_ref[...]
    b2 = b2_ref[...]
    for s in range(nsub):
        c0 = a0_ref[0, :, s:s + 1]
        c1 = a1_ref[0, :, s:s + 1]
        h = jnp.maximum(c0 * w10 + c1 * w11 + b1, 0.0)
        o_ref[s * 128:(s + 1) * 128, :] = (
            jax.lax.dot_general(h, w2, (((1,), (0,)), ((), ())),
                                preferred_element_type=jnp.float32)
            + b2
        )


def _mlp(a0, a1, W1, b1, W2, b2):
    grid, _, nsub = a0.shape
    rows_per_block = nsub * 128
    n = grid * rows_per_block
    return pl.pallas_call(
        functools.partial(_mlp_body, nsub),
        grid=(grid,),
        in_specs=[
            pl.BlockSpec((1, 128, nsub), lambda i: (i, 0, 0)),
            pl.BlockSpec((1, 128, nsub), lambda i: (i, 0, 0)),
            pl.BlockSpec((1, HID), lambda i: (0, 0)),
            pl.BlockSpec((1, HID), lambda i: (0, 0)),
            pl.BlockSpec((1, HID), lambda i: (0, 0)),
            pl.BlockSpec((HID, EMB), lambda i: (0, 0)),
            pl.BlockSpec((1, EMB), lambda i: (0, 0)),
        ],
        out_specs=pl.BlockSpec((rows_per_block, EMB), lambda i: (i, 0)),
        out_shape=jax.ShapeDtypeStruct((n, EMB), jnp.float32),
    )(a0, a1, W1[0:1, :], W1[1:2, :], b1.reshape(1, HID), W2,
      b2.reshape(1, EMB))


def _make_gather(total, chunk=128, nslot=4):
    per_w = total // _NW
    n_ch = per_w // chunk
    rounds = n_ch // nslot
    mesh = plsc.VectorSubcoreMesh(core_axis_name="c", subcore_axis_name="s")

    scratch = ([pltpu.VMEM((per_w,), jnp.int32)]
               + [pltpu.VMEM((chunk, EMB), jnp.float32)
                  for _ in range(nslot)]
               + [pltpu.SemaphoreType.DMA for _ in range(2 * nslot)])

    @functools.partial(
        pl.kernel,
        out_type=jax.ShapeDtypeStruct((total, EMB), jnp.float32),
        mesh=mesh,
        scratch_types=scratch,
    )
    def gather_k(idx_hbm, table_hbm, out_hbm, idx_v, *rest):
        bufs = rest[:nslot]
        gsems = rest[nslot:2 * nslot]
        ssems = rest[2 * nslot:]
        wid = lax.axis_index("s") * _NC + lax.axis_index("c")
        base = wid * per_w
        pltpu.sync_copy(idx_hbm.at[pl.ds(base, per_w)], idx_v)

        def g_start(c, j):
            return pltpu.async_copy(
                table_hbm.at[idx_v.at[pl.ds(c * chunk, chunk)]],
                bufs[j], gsems[j])

        def s_start(c, j):
            pltpu.async_copy(bufs[j],
                             out_hbm.at[pl.ds(base + c * chunk, chunk)],
                             ssems[j])

        def s_wait(c, j):
            pltpu.make_async_copy(
                bufs[j], out_hbm.at[pl.ds(base + c * chunk, chunk)],
                ssems[j]).wait()

        def body(k, carry):
            handles = []
            for j in range(nslot):
                c = k * nslot + j

                @pl.when(k > 0)
                def _():
                    s_wait(c - nslot, j)

                handles.append(g_start(c, j))
            for j in range(nslot):
                c = k * nslot + j
                handles[j].wait()
                s_start(c, j)

            return carry

        lax.fori_loop(0, rounds, body, 0)
        for j in range(nslot):
            s_wait(n_ch - nslot + j, j)

    return gather_k


def kernel(x, solutions, visited_time, W1, b1, W2, b2, pattern):
    bs, seq, nd = x.shape
    total = bs * seq
    idx = visited_time.reshape(total).astype(jnp.int32)
    rows_per_block = 16384
    grid = total // rows_per_block
    nsub = rows_per_block // 128
    a0 = x[:, :, 0].reshape(grid, nsub, 128).transpose(0, 2, 1)
    a1 = x[:, :, 1].reshape(grid, nsub, 128).transpose(0, 2, 1)
    pos = _make_gather(total)(idx, pattern).reshape(bs, seq, EMB)
    emb = _mlp(a0, a1, W1, b1, W2, b2).reshape(bs, seq, EMB)
    return (emb, pos)


# speedup vs baseline: 1.7996x; 1.4702x over previous
import functools

import jax
import jax.numpy as jnp
from jax import lax
from jax.experimental import pallas as pl
from jax.experimental.pallas import tpu as pltpu
from jax.experimental.pallas import tpu_sc as plsc

EMB = 128
HID = 64

_NC = 2
_NS = 16
_NW = _NC * _NS


def _mlp_body(nsub, a0_ref, a1_ref, w10_ref, w11_ref, b1_ref, w2_ref,
              b2_ref, o_ref):
    w10 = w10_ref[...]
    w11 = w11_ref[...]
    b1 = b1_ref[...]
    w2 = w2_ref[...]
    b2 = b2_ref[...]
    for s in range(nsub):
        c0 = a0_ref[0, :, s:s + 1]
        c1 = a1_ref[0, :, s:s + 1]
        h = jnp.maximum(c0 * w10 + c1 * w11 + b1, 0.0)
        o_ref[s * 128:(s + 1) * 128, :] = (
            jax.lax.dot_general(h, w2, (((1,), (0,)), ((), ())),
                                preferred_element_type=jnp.float32)
            + b2
        )


def _mlp(a0, a1, W1, b1, W2, b2):
    grid, _, nsub = a0.shape
    rows_per_block = nsub * 128
    n = grid * rows_per_block
    return pl.pallas_call(
        functools.partial(_mlp_body, nsub),
        grid=(grid,),
        in_specs=[
            pl.BlockSpec((1, 128, nsub), lambda i: (i, 0, 0)),
            pl.BlockSpec((1, 128, nsub), lambda i: (i, 0, 0)),
            pl.BlockSpec((1, HID), lambda i: (0, 0)),
            pl.BlockSpec((1, HID), lambda i: (0, 0)),
            pl.BlockSpec((1, HID), lambda i: (0, 0)),
            pl.BlockSpec((HID, EMB), lambda i: (0, 0)),
            pl.BlockSpec((1, EMB), lambda i: (0, 0)),
        ],
        out_specs=pl.BlockSpec((rows_per_block, EMB), lambda i: (i, 0)),
        out_shape=jax.ShapeDtypeStruct((n, EMB), jnp.float32),
    )(a0, a1, W1[0:1, :], W1[1:2, :], b1.reshape(1, HID), W2,
      b2.reshape(1, EMB))


def _make_gather(total, chunk=128, nslot=4):
    per_w = total // _NW
    n_ch = per_w // chunk
    rounds = n_ch // nslot
    mesh = plsc.VectorSubcoreMesh(core_axis_name="c", subcore_axis_name="s")

    scratch = ([pltpu.VMEM((per_w,), jnp.int32),
                pltpu.VMEM_SHARED((2048, EMB), jnp.float32)]
               + [pltpu.VMEM((chunk, EMB), jnp.float32)
                  for _ in range(nslot)]
               + [pltpu.SemaphoreType.DMA for _ in range(2 * nslot)])

    @functools.partial(
        pl.kernel,
        out_type=jax.ShapeDtypeStruct((total, EMB), jnp.float32),
        mesh=mesh,
        scratch_types=scratch,
    )
    def gather_k(idx_hbm, table_hbm, out_hbm, idx_v, tab_sp, *rest):
        bufs = rest[:nslot]
        gsems = rest[nslot:2 * nslot]
        ssems = rest[2 * nslot:]
        sid = lax.axis_index("s")
        wid = sid * _NC + lax.axis_index("c")
        base = wid * per_w
        rows_per_sub = 2048 // _NS
        pltpu.sync_copy(table_hbm.at[pl.ds(sid * rows_per_sub, rows_per_sub)],
                        tab_sp.at[pl.ds(sid * rows_per_sub, rows_per_sub)])
        pltpu.sync_copy(idx_hbm.at[pl.ds(base, per_w)], idx_v)
        plsc.subcore_barrier()

        def g_start(c, j):
            return pltpu.async_copy(
                tab_sp.at[idx_v.at[pl.ds(c * chunk, chunk)]],
                bufs[j], gsems[j])

        def s_start(c, j):
            pltpu.async_copy(bufs[j],
                             out_hbm.at[pl.ds(base + c * chunk, chunk)],
                             ssems[j])

        def s_wait(c, j):
            pltpu.make_async_copy(
                bufs[j], out_hbm.at[pl.ds(base + c * chunk, chunk)],
                ssems[j]).wait()

        def body(k, carry):
            handles = []
            for j in range(nslot):
                c = k * nslot + j

                @pl.when(k > 0)
                def _():
                    s_wait(c - nslot, j)

                handles.append(g_start(c, j))
            for j in range(nslot):
                c = k * nslot + j
                handles[j].wait()
                s_start(c, j)

            return carry

        lax.fori_loop(0, rounds, body, 0)
        for j in range(nslot):
            s_wait(n_ch - nslot + j, j)

    return gather_k


def kernel(x, solutions, visited_time, W1, b1, W2, b2, pattern):
    bs, seq, nd = x.shape
    total = bs * seq
    idx = visited_time.reshape(total).astype(jnp.int32)
    rows_per_block = 16384
    grid = total // rows_per_block
    nsub = rows_per_block // 128
    a0 = x[:, :, 0].reshape(grid, nsub, 128).transpose(0, 2, 1)
    a1 = x[:, :, 1].reshape(grid, nsub, 128).transpose(0, 2, 1)
    pos = _make_gather(total)(idx, pattern).reshape(bs, seq, EMB)
    emb = _mlp(a0, a1, W1, b1, W2, b2).reshape(bs, seq, EMB)
    return (emb, pos)


# speedup vs baseline: 1.8318x; 1.0179x over previous
import functools

import jax
import jax.numpy as jnp
from jax import lax
from jax.experimental import pallas as pl
from jax.experimental.pallas import tpu as pltpu
from jax.experimental.pallas import tpu_sc as plsc

EMB = 128
HID = 64

_NC = 2
_NS = 16
_NW = _NC * _NS


def _mlp_body(nsub, a0_ref, a1_ref, w10_ref, w11_ref, b1_ref, w2_ref,
              b2_ref, o_ref):
    w10 = w10_ref[...]
    w11 = w11_ref[...]
    b1 = b1_ref[...]
    w2 = w2_ref[...]
    b2 = b2_ref[...]
    for s in range(nsub):
        c0 = a0_ref[0, :, s:s + 1]
        c1 = a1_ref[0, :, s:s + 1]
        h = jnp.maximum(c0 * w10 + c1 * w11 + b1, 0.0)
        o_ref[s * 128:(s + 1) * 128, :] = (
            jax.lax.dot_general(h, w2, (((1,), (0,)), ((), ())),
                                preferred_element_type=jnp.float32)
            + b2
        )


def _mlp(a0, a1, W1, b1, W2, b2):
    grid, _, nsub = a0.shape
    rows_per_block = nsub * 128
    n = grid * rows_per_block
    return pl.pallas_call(
        functools.partial(_mlp_body, nsub),
        grid=(grid,),
        in_specs=[
            pl.BlockSpec((1, 128, nsub), lambda i: (i, 0, 0)),
            pl.BlockSpec((1, 128, nsub), lambda i: (i, 0, 0)),
            pl.BlockSpec((1, HID), lambda i: (0, 0)),
            pl.BlockSpec((1, HID), lambda i: (0, 0)),
            pl.BlockSpec((1, HID), lambda i: (0, 0)),
            pl.BlockSpec((HID, EMB), lambda i: (0, 0)),
            pl.BlockSpec((1, EMB), lambda i: (0, 0)),
        ],
        out_specs=pl.BlockSpec((rows_per_block, EMB), lambda i: (i, 0)),
        out_shape=jax.ShapeDtypeStruct((n, EMB), jnp.float32),
    )(a0, a1, W1[0:1, :], W1[1:2, :], b1.reshape(1, HID), W2,
      b2.reshape(1, EMB))


def _make_gather(total, chunk=128, nslot=4):
    per_w = total // _NW
    n_ch = per_w // chunk
    rounds = n_ch // nslot
    mesh = plsc.VectorSubcoreMesh(core_axis_name="c", subcore_axis_name="s")

    scratch = ([pltpu.VMEM((per_w,), jnp.int32),
                pltpu.VMEM_SHARED((2048, EMB), jnp.float32)]
               + [pltpu.VMEM((chunk, EMB), jnp.float32)
                  for _ in range(nslot)]
               + [pltpu.SemaphoreType.DMA for _ in range(2 * nslot)])

    @functools.partial(
        pl.kernel,
        out_type=jax.ShapeDtypeStruct((total, EMB), jnp.float32),
        mesh=mesh,
        scratch_types=scratch,
    )
    def gather_k(idx_hbm, table_hbm, out_hbm, idx_v, tab_sp, *rest):
        bufs = rest[:nslot]
        gsems = rest[nslot:2 * nslot]
        ssems = rest[2 * nslot:]
        sid = lax.axis_index("s")
        wid = sid * _NC + lax.axis_index("c")
        base = wid * per_w
        rows_per_sub = 2048 // _NS
        pltpu.sync_copy(table_hbm.at[pl.ds(sid * rows_per_sub, rows_per_sub)],
                        tab_sp.at[pl.ds(sid * rows_per_sub, rows_per_sub)])
        pltpu.sync_copy(idx_hbm.at[pl.ds(base, per_w)], idx_v)
        plsc.subcore_barrier()

        def g_start(c, j):
            return pltpu.async_copy(
                tab_sp.at[idx_v.at[pl.ds(c * chunk, chunk)]],
                bufs[j], gsems[j])

        def s_start(c, j):
            pltpu.async_copy(bufs[j],
                             out_hbm.at[pl.ds(base + c * chunk, chunk)],
                             ssems[j])

        def s_wait(c, j):
            pltpu.make_async_copy(
                bufs[j], out_hbm.at[pl.ds(base + c * chunk, chunk)],
                ssems[j]).wait()

        def body(k, carry):
            handles = []
            for j in range(nslot):
                c = k * nslot + j

                @pl.when(k > 0)
                def _():
                    s_wait(c - nslot, j)

                handles.append(g_start(c, j))
            for j in range(nslot):
                c = k * nslot + j
                handles[j].wait()
                s_start(c, j)

            return carry

        lax.fori_loop(0, rounds, body, 0)
        for j in range(nslot):
            s_wait(n_ch - nslot + j, j)

    return gather_k


def kernel(x, solutions, visited_time, W1, b1, W2, b2, pattern):
    bs, seq, nd = x.shape
    total = bs * seq
    idx = visited_time.reshape(total).astype(jnp.int32)
    rows_per_block = 32768
    grid = total // rows_per_block
    nsub = rows_per_block // 128
    a0 = x[:, :, 0].reshape(grid, nsub, 128).transpose(0, 2, 1)
    a1 = x[:, :, 1].reshape(grid, nsub, 128).transpose(0, 2, 1)
    pos = _make_gather(total)(idx, pattern).reshape(bs, seq, EMB)
    emb = _mlp(a0, a1, W1, b1, W2, b2).reshape(bs, seq, EMB)
    return (emb, pos)


# speedup vs baseline: 1.8783x; 1.0254x over previous
import functools

import jax
import jax.numpy as jnp
from jax import lax
from jax.experimental import pallas as pl
from jax.experimental.pallas import tpu as pltpu
from jax.experimental.pallas import tpu_sc as plsc

EMB = 128
HID = 64

_NC = 2
_NS = 16
_NW = _NC * _NS


def _mlp_body(nsub, grid, nbuf, a0_ref, a1_ref, w10_ref, w11_ref, b1_ref,
              w2_ref, b2_ref, o_hbm, *rest):
    bufs = rest[:nbuf]
    sems = rest[nbuf:]
    rows = nsub * 128
    w10 = w10_ref[...]
    w11 = w11_ref[...]
    b1 = b1_ref[...]
    w2 = w2_ref[...]
    b2 = b2_ref[...]
    i = pl.program_id(0)

    def compute_into(buf):
        for s in range(nsub):
            c0 = a0_ref[0, :, s:s + 1]
            c1 = a1_ref[0, :, s:s + 1]
            h = jnp.maximum(c0 * w10 + c1 * w11 + b1, 0.0)
            buf[s * 128:(s + 1) * 128, :] = (
                jax.lax.dot_general(h, w2, (((1,), (0,)), ((), ())),
                                    preferred_element_type=jnp.float32)
                + b2
            )

    for j in range(nbuf):
        @pl.when(lax.rem(i, nbuf) == j)
        def _():
            @pl.when(i >= nbuf)
            def _():
                pltpu.make_async_copy(
                    bufs[j], o_hbm.at[pl.ds((i - nbuf) * rows, rows)],
                    sems[j]).wait()

            compute_into(bufs[j])
            pltpu.make_async_copy(
                bufs[j], o_hbm.at[pl.ds(i * rows, rows)], sems[j]).start()

    @pl.when(i == grid - 1)
    def _():
        for j in range(nbuf):
            s_j = grid - 1 - ((grid - 1 - j) % nbuf)
            pltpu.make_async_copy(
                bufs[j], o_hbm.at[pl.ds(s_j * rows, rows)], sems[j]).wait()


def _mlp(a0, a1, W1, b1, W2, b2, nbuf=4):
    grid, _, nsub = a0.shape
    rows_per_block = nsub * 128
    n = grid * rows_per_block
    return pl.pallas_call(
        functools.partial(_mlp_body, nsub, grid, nbuf),
        grid=(grid,),
        in_specs=[
            pl.BlockSpec((1, 128, nsub), lambda i: (i, 0, 0)),
            pl.BlockSpec((1, 128, nsub), lambda i: (i, 0, 0)),
            pl.BlockSpec((1, HID), lambda i: (0, 0)),
            pl.BlockSpec((1, HID), lambda i: (0, 0)),
            pl.BlockSpec((1, HID), lambda i: (0, 0)),
            pl.BlockSpec((HID, EMB), lambda i: (0, 0)),
            pl.BlockSpec((1, EMB), lambda i: (0, 0)),
        ],
        out_specs=pl.BlockSpec(memory_space=pl.ANY),
        out_shape=jax.ShapeDtypeStruct((n, EMB), jnp.float32),
        scratch_shapes=(
            [pltpu.VMEM((rows_per_block, EMB), jnp.float32)
             for _ in range(nbuf)]
            + [pltpu.SemaphoreType.DMA for _ in range(nbuf)]),
    )(a0, a1, W1[0:1, :], W1[1:2, :], b1.reshape(1, HID), W2,
      b2.reshape(1, EMB))


def _make_gather(total, chunk=128, nslot=4):
    per_w = total // _NW
    n_ch = per_w // chunk
    rounds = n_ch // nslot
    mesh = plsc.VectorSubcoreMesh(core_axis_name="c", subcore_axis_name="s")

    scratch = ([pltpu.VMEM((per_w,), jnp.int32),
                pltpu.VMEM_SHARED((2048, EMB), jnp.float32)]
               + [pltpu.VMEM((chunk, EMB), jnp.float32)
                  for _ in range(nslot)]
               + [pltpu.SemaphoreType.DMA for _ in range(2 * nslot)])

    @functools.partial(
        pl.kernel,
        out_type=jax.ShapeDtypeStruct((total, EMB), jnp.float32),
        mesh=mesh,
        scratch_types=scratch,
    )
    def gather_k(idx_hbm, table_hbm, out_hbm, idx_v, tab_sp, *rest):
        bufs = rest[:nslot]
        gsems = rest[nslot:2 * nslot]
        ssems = rest[2 * nslot:]
        sid = lax.axis_index("s")
        wid = sid * _NC + lax.axis_index("c")
        base = wid * per_w
        rows_per_sub = 2048 // _NS
        pltpu.sync_copy(table_hbm.at[pl.ds(sid * rows_per_sub, rows_per_sub)],
                        tab_sp.at[pl.ds(sid * rows_per_sub, rows_per_sub)])
        pltpu.sync_copy(idx_hbm.at[pl.ds(base, per_w)], idx_v)
        plsc.subcore_barrier()

        def g_start(c, j):
            return pltpu.async_copy(
                tab_sp.at[idx_v.at[pl.ds(c * chunk, chunk)]],
                bufs[j], gsems[j])

        def s_start(c, j):
            pltpu.async_copy(bufs[j],
                             out_hbm.at[pl.ds(base + c * chunk, chunk)],
                             ssems[j])

        def s_wait(c, j):
            pltpu.make_async_copy(
                bufs[j], out_hbm.at[pl.ds(base + c * chunk, chunk)],
                ssems[j]).wait()

        def body(k, carry):
            handles = []
            for j in range(nslot):
                c = k * nslot + j

                @pl.when(k > 0)
                def _():
                    s_wait(c - nslot, j)

                handles.append(g_start(c, j))
            for j in range(nslot):
                c = k * nslot + j
                handles[j].wait()
                s_start(c, j)

            return carry

        lax.fori_loop(0, rounds, body, 0)
        for j in range(nslot):
            s_wait(n_ch - nslot + j, j)

    return gather_k


def kernel(x, solutions, visited_time, W1, b1, W2, b2, pattern):
    bs, seq, nd = x.shape
    total = bs * seq
    idx = visited_time.reshape(total).astype(jnp.int32)
    rows_per_block = 8192
    grid = total // rows_per_block
    nsub = rows_per_block // 128
    a0 = x[:, :, 0].reshape(grid, nsub, 128).transpose(0, 2, 1)
    a1 = x[:, :, 1].reshape(grid, nsub, 128).transpose(0, 2, 1)
    pos = _make_gather(total)(idx, pattern).reshape(bs, seq, EMB)
    emb = _mlp(a0, a1, W1, b1, W2, b2).reshape(bs, seq, EMB)
    return (emb, pos)


# speedup vs baseline: 2.0170x; 1.0739x over previous
import functools

import jax
import jax.numpy as jnp
from jax import lax
from jax.experimental import pallas as pl
from jax.experimental.pallas import tpu as pltpu
from jax.experimental.pallas import tpu_sc as plsc

EMB = 128
HID = 64

_NC = 2
_NS = 16
_NW = _NC * _NS


def _mlp_body(nsub, grid, nbuf, a0_ref, a1_ref, w10_ref, w11_ref, b1_ref,
              w2_ref, b2_ref, o_hbm, *rest):
    bufs = rest[:nbuf]
    sems = rest[nbuf:]
    rows = nsub * 128
    w10 = w10_ref[...]
    w11 = w11_ref[...]
    b1 = b1_ref[...]
    w2 = w2_ref[...]
    b2 = b2_ref[...]
    i = pl.program_id(0)

    w10b = w10.astype(jnp.bfloat16)
    w11b = w11.astype(jnp.bfloat16)
    b1b = b1.astype(jnp.bfloat16)
    w2b = w2.astype(jnp.bfloat16)

    def compute_into(buf):
        for s in range(nsub):
            c0 = a0_ref[0, :, s:s + 1].astype(jnp.bfloat16)
            c1 = a1_ref[0, :, s:s + 1].astype(jnp.bfloat16)
            h = jnp.maximum(c0 * w10b + c1 * w11b + b1b, 0.0)
            buf[s * 128:(s + 1) * 128, :] = (
                jax.lax.dot_general(h, w2b, (((1,), (0,)), ((), ())),
                                    preferred_element_type=jnp.float32)
                + b2
            )

    for j in range(nbuf):
        @pl.when(lax.rem(i, nbuf) == j)
        def _():
            @pl.when(i >= nbuf)
            def _():
                pltpu.make_async_copy(
                    bufs[j], o_hbm.at[pl.ds((i - nbuf) * rows, rows)],
                    sems[j]).wait()

            compute_into(bufs[j])
            pltpu.make_async_copy(
                bufs[j], o_hbm.at[pl.ds(i * rows, rows)], sems[j]).start()

    @pl.when(i == grid - 1)
    def _():
        for j in range(nbuf):
            s_j = grid - 1 - ((grid - 1 - j) % nbuf)
            pltpu.make_async_copy(
                bufs[j], o_hbm.at[pl.ds(s_j * rows, rows)], sems[j]).wait()


def _mlp(a0, a1, W1, b1, W2, b2, nbuf=4):
    grid, _, nsub = a0.shape
    rows_per_block = nsub * 128
    n = grid * rows_per_block
    return pl.pallas_call(
        functools.partial(_mlp_body, nsub, grid, nbuf),
        grid=(grid,),
        in_specs=[
            pl.BlockSpec((1, 128, nsub), lambda i: (i, 0, 0)),
            pl.BlockSpec((1, 128, nsub), lambda i: (i, 0, 0)),
            pl.BlockSpec((1, HID), lambda i: (0, 0)),
            pl.BlockSpec((1, HID), lambda i: (0, 0)),
            pl.BlockSpec((1, HID), lambda i: (0, 0)),
            pl.BlockSpec((HID, EMB), lambda i: (0, 0)),
            pl.BlockSpec((1, EMB), lambda i: (0, 0)),
        ],
        out_specs=pl.BlockSpec(memory_space=pl.ANY),
        out_shape=jax.ShapeDtypeStruct((n, EMB), jnp.float32),
        scratch_shapes=(
            [pltpu.VMEM((rows_per_block, EMB), jnp.float32)
             for _ in range(nbuf)]
            + [pltpu.SemaphoreType.DMA for _ in range(nbuf)]),
    )(a0, a1, W1[0:1, :], W1[1:2, :], b1.reshape(1, HID), W2,
      b2.reshape(1, EMB))


def _make_gather(total, chunk=128, nslot=4):
    per_w = total // _NW
    n_ch = per_w // chunk
    rounds = n_ch // nslot
    mesh = plsc.VectorSubcoreMesh(core_axis_name="c", subcore_axis_name="s")

    scratch = ([pltpu.VMEM((per_w,), jnp.int32),
                pltpu.VMEM_SHARED((2048, EMB), jnp.float32)]
               + [pltpu.VMEM((chunk, EMB), jnp.float32)
                  for _ in range(nslot)]
               + [pltpu.SemaphoreType.DMA for _ in range(2 * nslot)])

    @functools.partial(
        pl.kernel,
        out_type=jax.ShapeDtypeStruct((total, EMB), jnp.float32),
        mesh=mesh,
        scratch_types=scratch,
    )
    def gather_k(idx_hbm, table_hbm, out_hbm, idx_v, tab_sp, *rest):
        bufs = rest[:nslot]
        gsems = rest[nslot:2 * nslot]
        ssems = rest[2 * nslot:]
        sid = lax.axis_index("s")
        wid = sid * _NC + lax.axis_index("c")
        base = wid * per_w
        rows_per_sub = 2048 // _NS
        pltpu.sync_copy(table_hbm.at[pl.ds(sid * rows_per_sub, rows_per_sub)],
                        tab_sp.at[pl.ds(sid * rows_per_sub, rows_per_sub)])
        pltpu.sync_copy(idx_hbm.at[pl.ds(base, per_w)], idx_v)
        plsc.subcore_barrier()

        def g_start(c, j):
            return pltpu.async_copy(
                tab_sp.at[idx_v.at[pl.ds(c * chunk, chunk)]],
                bufs[j], gsems[j])

        def s_start(c, j):
            pltpu.async_copy(bufs[j],
                             out_hbm.at[pl.ds(base + c * chunk, chunk)],
                             ssems[j])

        def s_wait(c, j):
            pltpu.make_async_copy(
                bufs[j], out_hbm.at[pl.ds(base + c * chunk, chunk)],
                ssems[j]).wait()

        def body(k, carry):
            handles = []
            for j in range(nslot):
                c = k * nslot + j

                @pl.when(k > 0)
                def _():
                    s_wait(c - nslot, j)

                handles.append(g_start(c, j))
            for j in range(nslot):
                c = k * nslot + j
                handles[j].wait()
                s_start(c, j)

            return carry

        lax.fori_loop(0, rounds, body, 0)
        for j in range(nslot):
            s_wait(n_ch - nslot + j, j)

    return gather_k


def kernel(x, solutions, visited_time, W1, b1, W2, b2, pattern):
    bs, seq, nd = x.shape
    total = bs * seq
    idx = visited_time.reshape(total).astype(jnp.int32)
    rows_per_block = 8192
    grid = total // rows_per_block
    nsub = rows_per_block // 128
    a0 = x[:, :, 0].reshape(grid, nsub, 128).transpose(0, 2, 1)
    a1 = x[:, :, 1].reshape(grid, nsub, 128).transpose(0, 2, 1)
    pos = _make_gather(total)(idx, pattern).reshape(bs, seq, EMB)
    emb = _mlp(a0, a1, W1, b1, W2, b2).reshape(bs, seq, EMB)
    return (emb, pos)


# speedup vs baseline: 2.0309x; 1.0069x over previous
import functools

import jax
import jax.numpy as jnp
from jax import lax
from jax.experimental import pallas as pl
from jax.experimental.pallas import tpu as pltpu
from jax.experimental.pallas import tpu_sc as plsc

EMB = 128
HID = 64

_NC = 2
_NS = 16
_NW = _NC * _NS


def _mlp_body(nsub, grid, nbuf, a0_ref, a1_ref, w10_ref, w11_ref, b1_ref,
              w2_ref, b2_ref, o_hbm, *rest):
    bufs = rest[:nbuf]
    sems = rest[nbuf:]
    rows = nsub * 128
    w10 = w10_ref[...]
    w11 = w11_ref[...]
    b1 = b1_ref[...]
    w2 = w2_ref[...]
    b2 = b2_ref[...]
    i = pl.program_id(0)

    w10b = w10.astype(jnp.bfloat16)
    w11b = w11.astype(jnp.bfloat16)
    b1b = b1.astype(jnp.bfloat16)
    w2b = w2.astype(jnp.bfloat16)

    def compute_into(buf):
        for s in range(nsub):
            c0 = a0_ref[0, 0, :, s:s + 1].astype(jnp.bfloat16)
            c1 = a1_ref[0, 0, :, s:s + 1].astype(jnp.bfloat16)
            h = jnp.maximum(c0 * w10b + c1 * w11b + b1b, 0.0)
            buf[s * 128:(s + 1) * 128, :] = (
                jax.lax.dot_general(h, w2b, (((1,), (0,)), ((), ())),
                                    preferred_element_type=jnp.float32)
                + b2
            )

    for j in range(nbuf):
        @pl.when(lax.rem(i, nbuf) == j)
        def _():
            @pl.when(i >= nbuf)
            def _():
                pltpu.make_async_copy(
                    bufs[j], o_hbm.at[pl.ds((i - nbuf) * rows, rows)],
                    sems[j]).wait()

            compute_into(bufs[j])
            pltpu.make_async_copy(
                bufs[j], o_hbm.at[pl.ds(i * rows, rows)], sems[j]).start()

    @pl.when(i == grid - 1)
    def _():
        for j in range(nbuf):
            s_j = grid - 1 - ((grid - 1 - j) % nbuf)
            pltpu.make_async_copy(
                bufs[j], o_hbm.at[pl.ds(s_j * rows, rows)], sems[j]).wait()


def _mlp(a0, a1, W1, b1, W2, b2, nbuf=4):
    _, grid, _, nsub = a0.shape
    rows_per_block = nsub * 128
    n = grid * rows_per_block
    return pl.pallas_call(
        functools.partial(_mlp_body, nsub, grid, nbuf),
        grid=(grid,),
        in_specs=[
            pl.BlockSpec((1, 1, 128, nsub), lambda i: (0, i, 0, 0)),
            pl.BlockSpec((1, 1, 128, nsub), lambda i: (1, i, 0, 0)),
            pl.BlockSpec((1, HID), lambda i: (0, 0)),
            pl.BlockSpec((1, HID), lambda i: (0, 0)),
            pl.BlockSpec((1, HID), lambda i: (0, 0)),
            pl.BlockSpec((HID, EMB), lambda i: (0, 0)),
            pl.BlockSpec((1, EMB), lambda i: (0, 0)),
        ],
        out_specs=pl.BlockSpec(memory_space=pl.ANY),
        out_shape=jax.ShapeDtypeStruct((n, EMB), jnp.float32),
        scratch_shapes=(
            [pltpu.VMEM((rows_per_block, EMB), jnp.float32)
             for _ in range(nbuf)]
            + [pltpu.SemaphoreType.DMA for _ in range(nbuf)]),
    )(a0, a1, W1[0:1, :], W1[1:2, :], b1.reshape(1, HID), W2,
      b2.reshape(1, EMB))


def _make_gather(total, chunk=128, nslot=4):
    per_w = total // _NW
    n_ch = per_w // chunk
    rounds = n_ch // nslot
    mesh = plsc.VectorSubcoreMesh(core_axis_name="c", subcore_axis_name="s")

    scratch = ([pltpu.VMEM((per_w,), jnp.int32),
                pltpu.VMEM_SHARED((2048, EMB), jnp.float32)]
               + [pltpu.VMEM((chunk, EMB), jnp.float32)
                  for _ in range(nslot)]
               + [pltpu.SemaphoreType.DMA for _ in range(2 * nslot)])

    @functools.partial(
        pl.kernel,
        out_type=jax.ShapeDtypeStruct((total, EMB), jnp.float32),
        mesh=mesh,
        scratch_types=scratch,
    )
    def gather_k(idx_hbm, table_hbm, out_hbm, idx_v, tab_sp, *rest):
        bufs = rest[:nslot]
        gsems = rest[nslot:2 * nslot]
        ssems = rest[2 * nslot:]
        sid = lax.axis_index("s")
        wid = sid * _NC + lax.axis_index("c")
        base = wid * per_w
        rows_per_sub = 2048 // _NS
        pltpu.sync_copy(table_hbm.at[pl.ds(sid * rows_per_sub, rows_per_sub)],
                        tab_sp.at[pl.ds(sid * rows_per_sub, rows_per_sub)])
        pltpu.sync_copy(idx_hbm.at[pl.ds(base, per_w)], idx_v)
        plsc.subcore_barrier()

        def g_start(c, j):
            return pltpu.async_copy(
                tab_sp.at[idx_v.at[pl.ds(c * chunk, chunk)]],
                bufs[j], gsems[j])

        def s_start(c, j):
            pltpu.async_copy(bufs[j],
                             out_hbm.at[pl.ds(base + c * chunk, chunk)],
                             ssems[j])

        def s_wait(c, j):
            pltpu.make_async_copy(
                bufs[j], out_hbm.at[pl.ds(base + c * chunk, chunk)],
                ssems[j]).wait()

        def body(k, carry):
            handles = []
            for j in range(nslot):
                c = k * nslot + j

                @pl.when(k > 0)
                def _():
                    s_wait(c - nslot, j)

                handles.append(g_start(c, j))
            for j in range(nslot):
                c = k * nslot + j
                handles[j].wait()
                s_start(c, j)

            return carry

        lax.fori_loop(0, rounds, body, 0)
        for j in range(nslot):
            s_wait(n_ch - nslot + j, j)

    return gather_k


def kernel(x, solutions, visited_time, W1, b1, W2, b2, pattern):
    bs, seq, nd = x.shape
    total = bs * seq
    idx = visited_time.reshape(total).astype(jnp.int32)
    rows_per_block = 8192
    grid = total // rows_per_block
    nsub = rows_per_block // 128
    a = x.reshape(grid, nsub, 128, 2).transpose(3, 0, 2, 1)
    pos = _make_gather(total)(idx, pattern).reshape(bs, seq, EMB)
    emb = _mlp(a, a, W1, b1, W2, b2).reshape(bs, seq, EMB)
    return (emb, pos)
